# Initial kernel scaffold; baseline (speedup 1.0000x reference)
import jax
import jax.numpy as jnp
from jax.experimental import pallas as pl


def kernel(loc_data, conf_data, priors, targets):
    raise NotImplementedError("write your pallas kernel here")


# speedup vs baseline: 64.5153x; 64.5153x over previous
import functools

import jax
import jax.numpy as jnp
from jax.experimental import pallas as pl
from jax.experimental.pallas import tpu as pltpu

_NEGPOS = 3
_VAR0 = 0.1
_VAR1 = 0.2
_THR = 0.35
_L = 128


def _mbl_body(truths_ref, loc_ref, conf_ref, pri_ref, out_ref, *, P, R, O):
    b = pl.program_id(0)
    L = _L

    pcx = pri_ref[0]
    pcy = pri_ref[1]
    pw = pri_ref[2]
    ph = pri_ref[3]
    px1 = pcx - pw / 2
    py1 = pcy - ph / 2
    px2 = pcx + pw / 2
    py2 = pcy + ph / 2
    area_p = (px2 - px1) * (py2 - py1)

    row_i = jax.lax.broadcasted_iota(jnp.int32, (R, L), 0)
    lane_i = jax.lax.broadcasted_iota(jnp.int32, (R, L), 1)
    flat = row_i * L + lane_i
    real = flat < P

    best_ov = jnp.full((R, L), -1.0, jnp.float32)
    best_idx = jnp.zeros((R, L), jnp.int32)
    bps = []
    txs = []
    for t in range(O):
        tx1 = truths_ref[b, 4 * t + 0]
        ty1 = truths_ref[b, 4 * t + 1]
        tx2 = truths_ref[b, 4 * t + 2]
        ty2 = truths_ref[b, 4 * t + 3]
        txs.append((tx1, ty1, tx2, ty2))
        ix = jnp.maximum(jnp.minimum(tx2, px2) - jnp.maximum(tx1, px1), 0.0)
        iy = jnp.maximum(jnp.minimum(ty2, py2) - jnp.maximum(ty1, py1), 0.0)
        inter = ix * iy
        area_t = (tx2 - tx1) * (ty2 - ty1)
        iou = inter / (area_t + area_p - inter)
        iou = jnp.where(real, iou, -2.0)
        upd = iou > best_ov
        best_idx = jnp.where(upd, t, best_idx)
        best_ov = jnp.where(upd, iou, best_ov)
        m = jnp.max(iou)
        bps.append(jnp.min(jnp.where(iou == m, flat, jnp.int32(0x7FFFFFFF))))

    for t in range(O):
        hit = flat == bps[t]
        best_ov = jnp.where(hit, 2.0, best_ov)
        best_idx = jnp.where(hit, t, best_idx)

    pos = best_ov >= _THR
    np_i = jnp.sum(pos.astype(jnp.int32))

    mx1 = jnp.zeros((R, L), jnp.float32)
    my1 = jnp.zeros((R, L), jnp.float32)
    mx2 = jnp.zeros((R, L), jnp.float32)
    my2 = jnp.zeros((R, L), jnp.float32)
    for t in range(O):
        sel = best_idx == t
        tx1, ty1, tx2, ty2 = txs[t]
        mx1 = jnp.where(sel, tx1, mx1)
        my1 = jnp.where(sel, ty1, my1)
        mx2 = jnp.where(sel, tx2, mx2)
        my2 = jnp.where(sel, ty2, my2)

    g_cx = ((mx1 + mx2) / 2 - pcx) / (_VAR0 * pw)
    g_cy = ((my1 + my2) / 2 - pcy) / (_VAR0 * ph)
    g_w = jnp.log((mx2 - mx1) / pw) / _VAR1
    g_h = jnp.log((my2 - my1) / ph) / _VAR1

    def sl1(pred, g):
        ad = jnp.abs(pred - g)
        return jnp.where(ad < 1.0, 0.5 * ad * ad, ad - 0.5)

    lsum = (sl1(loc_ref[0, 0], g_cx) + sl1(loc_ref[0, 1], g_cy)
            + sl1(loc_ref[0, 2], g_w) + sl1(loc_ref[0, 3], g_h))
    loss_l = jnp.sum(jnp.where(pos, lsum, 0.0))

    c0 = conf_ref[0, 0]
    c1 = conf_ref[0, 1]
    cm = jnp.maximum(c0, c1)
    lse = cm + jnp.log(jnp.exp(c0 - cm) + jnp.exp(c1 - cm))
    ce = lse - jnp.where(pos, c1, c0)
    pos_sum = jnp.sum(jnp.where(pos, ce, 0.0))

    nv = jnp.where(pos | (~real), -1.0, ce)
    nvi = jax.lax.bitcast_convert_type(nv, jnp.int32)
    k = jnp.minimum(_NEGPOS * np_i, P - 1)
    k = jnp.minimum(k, P - np_i)
    thr = jnp.int32(0)
    for bit in range(30, -1, -1):
        cand = thr | jnp.int32(1 << bit)
        cnt = jnp.sum((nvi >= cand).astype(jnp.int32))
        thr = jnp.where(cnt >= k, cand, thr)
    cnt_gt = jnp.sum((nvi > thr).astype(jnp.int32))
    sum_gt = jnp.sum(jnp.where(nvi > thr, nv, 0.0))
    thr_f = jax.lax.bitcast_convert_type(thr, jnp.float32)
    neg_sum = jnp.where(
        k > 0, sum_gt + (k - cnt_gt).astype(jnp.float32) * thr_f, 0.0)

    r8 = jax.lax.broadcasted_iota(jnp.int32, (8, L), 0)
    l8 = jax.lax.broadcasted_iota(jnp.int32, (8, L), 1)
    o = jnp.zeros((8, L), jnp.float32)
    o = jnp.where((r8 == 0) & (l8 == 0), np_i.astype(jnp.float32), o)
    o = jnp.where((r8 == 0) & (l8 == 1), loss_l, o)
    o = jnp.where((r8 == 0) & (l8 == 2), pos_sum, o)
    o = jnp.where((r8 == 0) & (l8 == 3), neg_sum, o)
    out_ref[0] = o


@jax.jit
def kernel(loc_data, conf_data, priors, targets):
    B, P, _ = loc_data.shape
    O = targets.shape[1]
    L = _L
    PP = ((P + 8 * L - 1) // (8 * L)) * (8 * L)
    R = PP // L
    pad = PP - P

    locp = jnp.pad(loc_data, ((0, 0), (0, pad), (0, 0)))
    locp = locp.transpose(0, 2, 1).reshape(B, 4, R, L)
    confp = jnp.pad(conf_data, ((0, 0), (0, pad), (0, 0)))
    confp = confp.transpose(0, 2, 1).reshape(B, 2, R, L)
    pad_pri = jnp.tile(jnp.array([[0.0, 0.0, 1.0, 1.0]], jnp.float32),
                       (pad, 1))
    prip = jnp.concatenate([priors, pad_pri], axis=0).T.reshape(4, R, L)
    truths2 = targets[..., :4].reshape(B, 4 * O)

    out = pl.pallas_call(
        functools.partial(_mbl_body, P=P, R=R, O=O),
        grid=(B,),
        in_specs=[
            pl.BlockSpec(memory_space=pltpu.SMEM),
            pl.BlockSpec((1, 4, R, L), lambda b: (b, 0, 0, 0)),
            pl.BlockSpec((1, 2, R, L), lambda b: (b, 0, 0, 0)),
            pl.BlockSpec((4, R, L), lambda b: (0, 0, 0)),
        ],
        out_specs=pl.BlockSpec((1, 8, L), lambda b: (b, 0, 0)),
        out_shape=jax.ShapeDtypeStruct((B, 8, L), jnp.float32),
        compiler_params=pltpu.CompilerParams(
            dimension_semantics=("arbitrary",)),
    )(truths2, locp, confp, prip)

    npos = out[:, 0, 0]
    n_total = jnp.maximum(jnp.sum(npos), 1.0)
    loss_l = jnp.sum(out[:, 0, 1]) / n_total
    loss_c = (jnp.sum(out[:, 0, 2]) + jnp.sum(out[:, 0, 3])) / n_total
    return loss_l, loss_c


# speedup vs baseline: 66.7186x; 1.0342x over previous
import functools

import jax
import jax.numpy as jnp
from jax.experimental import pallas as pl
from jax.experimental.pallas import tpu as pltpu

_NEGPOS = 3
_VAR0 = 0.1
_VAR1 = 0.2
_THR = 0.35
_L = 128


def _mbl_body(truths_ref, loc_ref, conf_ref, pri_ref, out_ref, *, P, R, O, BR):
    bb = pl.program_id(0)
    L = _L

    pcx = pri_ref[0]
    pcy = pri_ref[1]
    pw = pri_ref[2]
    ph = pri_ref[3]
    px1 = pcx - pw / 2
    py1 = pcy - ph / 2
    px2 = pcx + pw / 2
    py2 = pcy + ph / 2
    area_p = (px2 - px1) * (py2 - py1)

    row_i = jax.lax.broadcasted_iota(jnp.int32, (R, L), 0)
    lane_i = jax.lax.broadcasted_iota(jnp.int32, (R, L), 1)
    flat = row_i * L + lane_i

    for rr in range(BR):
        b = bb * BR + rr
        best_ov = jnp.full((R, L), -1.0, jnp.float32)
        best_idx = jnp.zeros((R, L), jnp.int32)
        bps = []
        txs = []
        for t in range(O):
            tx1 = truths_ref[b, 4 * t + 0]
            ty1 = truths_ref[b, 4 * t + 1]
            tx2 = truths_ref[b, 4 * t + 2]
            ty2 = truths_ref[b, 4 * t + 3]
            txs.append((tx1, ty1, tx2, ty2))
            ix = jnp.maximum(jnp.minimum(tx2, px2) - jnp.maximum(tx1, px1),
                             0.0)
            iy = jnp.maximum(jnp.minimum(ty2, py2) - jnp.maximum(ty1, py1),
                             0.0)
            inter = ix * iy
            area_t = (tx2 - tx1) * (ty2 - ty1)
            iou = inter / (area_t + area_p - inter)
            upd = iou > best_ov
            best_idx = jnp.where(upd, t, best_idx)
            best_ov = jnp.where(upd, iou, best_ov)
            m = jnp.max(iou)
            bps.append(jnp.min(jnp.where(iou == m, flat,
                                         jnp.int32(0x7FFFFFFF))))

        for t in range(O):
            hit = flat == bps[t]
            best_ov = jnp.where(hit, 2.0, best_ov)
            best_idx = jnp.where(hit, t, best_idx)

        pos = best_ov >= _THR
        np_i = jnp.sum(pos.astype(jnp.int32))

        mx1 = jnp.zeros((R, L), jnp.float32)
        my1 = jnp.zeros((R, L), jnp.float32)
        mx2 = jnp.zeros((R, L), jnp.float32)
        my2 = jnp.zeros((R, L), jnp.float32)
        for t in range(O):
            sel = best_idx == t
            tx1, ty1, tx2, ty2 = txs[t]
            mx1 = jnp.where(sel, tx1, mx1)
            my1 = jnp.where(sel, ty1, my1)
            mx2 = jnp.where(sel, tx2, mx2)
            my2 = jnp.where(sel, ty2, my2)

        g_cx = ((mx1 + mx2) / 2 - pcx) / (_VAR0 * pw)
        g_cy = ((my1 + my2) / 2 - pcy) / (_VAR0 * ph)
        g_w = jnp.log((mx2 - mx1) / pw) / _VAR1
        g_h = jnp.log((my2 - my1) / ph) / _VAR1

        def sl1(pred, g):
            ad = jnp.abs(pred - g)
            return jnp.where(ad < 1.0, 0.5 * ad * ad, ad - 0.5)

        lsum = (sl1(loc_ref[rr, 0], g_cx) + sl1(loc_ref[rr, 1], g_cy)
                + sl1(loc_ref[rr, 2], g_w) + sl1(loc_ref[rr, 3], g_h))
        loss_l = jnp.sum(jnp.where(pos, lsum, 0.0))

        c0 = conf_ref[rr, 0]
        c1 = conf_ref[rr, 1]
        cm = jnp.maximum(c0, c1)
        lse = cm + jnp.log(jnp.exp(c0 - cm) + jnp.exp(c1 - cm))
        ce = lse - jnp.where(pos, c1, c0)
        pos_sum = jnp.sum(jnp.where(pos, ce, 0.0))

        nv = jnp.where(pos, -1.0, ce)
        nvi = jax.lax.bitcast_convert_type(nv, jnp.int32)
        k = jnp.minimum(_NEGPOS * np_i, P - 1)
        k = jnp.minimum(k, P - np_i)
        thr = jnp.int32(0)
        for bit in range(30, -1, -1):
            cand = thr | jnp.int32(1 << bit)
            cnt = jnp.sum((nvi >= cand).astype(jnp.int32))
            thr = jnp.where(cnt >= k, cand, thr)
        cnt_gt = jnp.sum((nvi > thr).astype(jnp.int32))
        sum_gt = jnp.sum(jnp.where(nvi > thr, nv, 0.0))
        thr_f = jax.lax.bitcast_convert_type(thr, jnp.float32)
        neg_sum = jnp.where(
            k > 0, sum_gt + (k - cnt_gt).astype(jnp.float32) * thr_f, 0.0)

        r8 = jax.lax.broadcasted_iota(jnp.int32, (8, L), 0)
        l8 = jax.lax.broadcasted_iota(jnp.int32, (8, L), 1)
        o = jnp.zeros((8, L), jnp.float32)
        o = jnp.where((r8 == 0) & (l8 == 0), np_i.astype(jnp.float32), o)
        o = jnp.where((r8 == 0) & (l8 == 1), loss_l, o)
        o = jnp.where((r8 == 0) & (l8 == 2), pos_sum, o)
        o = jnp.where((r8 == 0) & (l8 == 3), neg_sum, o)
        out_ref[rr] = o


@jax.jit
def kernel(loc_data, conf_data, priors, targets):
    B, P, _ = loc_data.shape
    O = targets.shape[1]
    L = _L
    PP = ((P + 8 * L - 1) // (8 * L)) * (8 * L)
    R = PP // L
    pad = PP - P

    BR = 2 if B % 2 == 0 else 1

    locp = jnp.pad(loc_data, ((0, 0), (0, pad), (0, 0)))
    locp = locp.transpose(0, 2, 1).reshape(B, 4, R, L)
    conf_pad = jnp.broadcast_to(jnp.array([100.0, -100.0], jnp.float32),
                                (B, pad, 2))
    confp = jnp.concatenate([conf_data, conf_pad], axis=1)
    confp = confp.transpose(0, 2, 1).reshape(B, 2, R, L)
    pad_pri = jnp.tile(jnp.array([[3.0, 3.0, 1.0, 1.0]], jnp.float32),
                       (pad, 1))
    prip = jnp.concatenate([priors, pad_pri], axis=0).T.reshape(4, R, L)
    truths2 = targets[..., :4].reshape(B, 4 * O)

    out = pl.pallas_call(
        functools.partial(_mbl_body, P=P, R=R, O=O, BR=BR),
        grid=(B // BR,),
        in_specs=[
            pl.BlockSpec(memory_space=pltpu.SMEM),
            pl.BlockSpec((BR, 4, R, L), lambda b: (b, 0, 0, 0)),
            pl.BlockSpec((BR, 2, R, L), lambda b: (b, 0, 0, 0)),
            pl.BlockSpec((4, R, L), lambda b: (0, 0, 0)),
        ],
        out_specs=pl.BlockSpec((BR, 8, L), lambda b: (b, 0, 0)),
        out_shape=jax.ShapeDtypeStruct((B, 8, L), jnp.float32),
        compiler_params=pltpu.CompilerParams(
            dimension_semantics=("arbitrary",)),
    )(truths2, locp, confp, prip)

    npos = out[:, 0, 0]
    n_total = jnp.maximum(jnp.sum(npos), 1.0)
    loss_l = jnp.sum(out[:, 0, 1]) / n_total
    loss_c = (jnp.sum(out[:, 0, 2]) + jnp.sum(out[:, 0, 3])) / n_total
    return loss_l, loss_c
